# baseline (speedup 1.0000x reference)
import functools

import jax
import jax.numpy as jnp
from jax import lax
from jax.experimental import pallas as pl
from jax.experimental.pallas import tpu as pltpu
from jax.experimental.pallas import tpu_sc as plsc

_CAPACITY = 0.5
_BLK_T = 256
_T_SC = 3072
_C = 8
_LANES = 16


def _moments_kernel(a_ref, p_ref, dst_ref, dch_ref, *, inv_d):
    a = a_ref[...]
    p = p_ref[...]
    d = a - p
    dst_ref[...] = jnp.sum(a * a, axis=-1) * inv_d
    dch_ref[...] = jnp.sum(d * d, axis=-1) * inv_d


def _sc_moments_body(a_hbm, p_hbm, dst_hbm, dch_hbm,
                     a0, p0, a1, p1, dstb, dchb, sem0, sem1,
                     *, t_full, t_tc, t_sc, d):
    nc = 2
    wid = lax.axis_index("s") * nc + lax.axis_index("c")
    tiles_per_batch = 8
    q = t_sc // tiles_per_batch
    b = wid // tiles_per_batch
    w8 = wid % tiles_per_batch
    row_base = b * t_full + t_tc + w8 * q
    out_base = b * t_sc + w8 * q
    n_chunks = q // _C
    n_col = d // _LANES

    def start(chunk, ab, pb, sem):
        src = a_hbm.at[pl.ds(row_base + chunk * _C, _C)]
        pltpu.make_async_copy(src, ab, sem).start()
        srcp = p_hbm.at[pl.ds(row_base + chunk * _C, _C)]
        pltpu.make_async_copy(srcp, pb, sem).start()

    def drain(chunk, ab, pb, sem):
        src = a_hbm.at[pl.ds(row_base + chunk * _C, _C)]
        pltpu.make_async_copy(src, ab, sem).wait()
        srcp = p_hbm.at[pl.ds(row_base + chunk * _C, _C)]
        pltpu.make_async_copy(srcp, pb, sem).wait()

    start(0, a0, p0, sem0)
    start(1, a1, p1, sem1)

    zero16 = jnp.zeros((_LANES,), jnp.float32)

    def chunk_pair(it, _):
        c = it * 2
        for par, (ab, pb, sem) in enumerate(((a0, p0, sem0),
                                             (a1, p1, sem1))):
            chunk = c + par
            drain(chunk, ab, pb, sem)

            def mbody(m, accs):
                out = []
                for j in range(_C):
                    va = ab[j, pl.ds(m * _LANES, _LANES)]
                    vp = pb[j, pl.ds(m * _LANES, _LANES)]
                    dv = va - vp
                    out.append(accs[2 * j] + va * va)
                    out.append(accs[2 * j + 1] + dv * dv)
                return tuple(out)

            accs = lax.fori_loop(0, n_col, mbody,
                                 tuple(zero16 for _ in range(2 * _C)))
            for j in range(_C):
                dstb[pl.ds((chunk * _C + j) * _LANES, _LANES)] = accs[2 * j]
                dchb[pl.ds((chunk * _C + j) * _LANES, _LANES)] = accs[2 * j + 1]

            @pl.when(chunk + 2 < n_chunks)
            def _():
                start(chunk + 2, ab, pb, sem)

        return 0

    lax.fori_loop(0, n_chunks // 2, chunk_pair, 0)

    pltpu.sync_copy(dstb, dst_hbm.at[pl.ds(out_base * _LANES, q * _LANES)])
    pltpu.sync_copy(dchb, dch_hbm.at[pl.ds(out_base * _LANES, q * _LANES)])


def _select_topk_mask(g, k):
    b, t = g.shape
    bits = jax.lax.bitcast_convert_type(g, jnp.int32)

    def vbody(_, carry):
        lo, hi = carry
        mid = lo + ((hi - lo) >> 1)
        cnt = jnp.sum((bits >= mid).astype(jnp.int32), axis=1, keepdims=True)
        feas = cnt >= k
        return jnp.where(feas, mid, lo), jnp.where(feas, hi, mid)

    lo0 = jnp.zeros((b, 1), jnp.int32)
    hi0 = jnp.full((b, 1), jnp.int32(0x40000001))
    tbits, _ = jax.lax.fori_loop(0, 31, vbody, (lo0, hi0))

    gt = bits > tbits
    eq = bits == tbits
    need = k - jnp.sum(gt.astype(jnp.int32), axis=1, keepdims=True)
    iota = jax.lax.broadcasted_iota(jnp.int32, (b, t), 1)
    eqi = eq.astype(jnp.int32)

    def ibody(_, carry):
        lo, hi = carry
        mid = lo + ((hi - lo) >> 1)
        cnt = jnp.sum(eqi * (iota < mid).astype(jnp.int32), axis=1,
                      keepdims=True)
        geq = cnt >= need
        return jnp.where(geq, lo, mid), jnp.where(geq, mid, hi)

    lo0 = jnp.zeros((b, 1), jnp.int32)
    hi0 = jnp.full((b, 1), jnp.int32(t))
    _, jstar = jax.lax.fori_loop(0, 14, ibody, (lo0, hi0))

    return (gt | (eq & (iota < jstar))).astype(jnp.float32)


def _gate_kernel(scal_ref, dst_tc_ref, dch_tc_ref, dstp_ref, dchp_ref,
                 g_ref, bin_ref, *, k, inv_d):
    dst_sc = jnp.sum(dstp_ref[...], axis=-1) * inv_d
    dch_sc = jnp.sum(dchp_ref[...], axis=-1) * inv_d
    dst = jnp.concatenate([dst_tc_ref[...], dst_sc], axis=1)
    dch = jnp.concatenate([dch_tc_ref[...], dch_sc], axis=1)
    log_oce = scal_ref[0]
    m_cu = scal_ref[1]
    bce_pos = scal_ref[2]
    bcu_pos = scal_ref[3]

    ce = dst - (dch - log_oce)
    ma = jnp.mean(dst)
    cu = dst - m_cu * ma
    s_ce = jax.nn.sigmoid(bce_pos * ce)
    s_cu = jax.nn.sigmoid(bcu_pos * cu)
    g = s_ce + s_cu - s_ce * s_cu
    g_ref[...] = g
    bin_ref[...] = _select_topk_mask(g, k)


def kernel(actual_residual, predicted_residual, o_ce, m_cu, beta_ce, beta_cu):
    bv, tv, dv = actual_residual.shape
    k = max(1, int(tv * _CAPACITY))
    t_sc = _T_SC
    t_tc = tv - t_sc

    dst_tc, dch_tc = pl.pallas_call(
        functools.partial(_moments_kernel, inv_d=1.0 / dv),
        grid=(t_tc // _BLK_T,),
        in_specs=[
            pl.BlockSpec((bv, _BLK_T, dv), lambda i: (0, i, 0)),
            pl.BlockSpec((bv, _BLK_T, dv), lambda i: (0, i, 0)),
        ],
        out_specs=[
            pl.BlockSpec((bv, _BLK_T), lambda i: (0, i)),
            pl.BlockSpec((bv, _BLK_T), lambda i: (0, i)),
        ],
        out_shape=[
            jax.ShapeDtypeStruct((bv, t_tc), jnp.float32),
            jax.ShapeDtypeStruct((bv, t_tc), jnp.float32),
        ],
        compiler_params=pltpu.CompilerParams(
            dimension_semantics=("parallel",),
        ),
    )(actual_residual, predicted_residual)

    n_sc = bv * t_sc
    q = t_sc // 8
    a2 = actual_residual.reshape(bv * tv, dv)
    p2 = predicted_residual.reshape(bv * tv, dv)
    sc_fn = pl.kernel(
        functools.partial(_sc_moments_body, t_full=tv, t_tc=t_tc,
                          t_sc=t_sc, d=dv),
        out_type=[
            jax.ShapeDtypeStruct((n_sc * _LANES,), jnp.float32),
            jax.ShapeDtypeStruct((n_sc * _LANES,), jnp.float32),
        ],
        mesh=plsc.VectorSubcoreMesh(core_axis_name="c", subcore_axis_name="s"),
        scratch_types=[
            pltpu.VMEM((_C, dv), jnp.float32),
            pltpu.VMEM((_C, dv), jnp.float32),
            pltpu.VMEM((_C, dv), jnp.float32),
            pltpu.VMEM((_C, dv), jnp.float32),
            pltpu.VMEM((q * _LANES,), jnp.float32),
            pltpu.VMEM((q * _LANES,), jnp.float32),
            pltpu.SemaphoreType.DMA,
            pltpu.SemaphoreType.DMA,
        ],
    )
    dst_p, dch_p = sc_fn(a2, p2)
    dst_p = dst_p.reshape(bv, t_sc, _LANES)
    dch_p = dch_p.reshape(bv, t_sc, _LANES)

    scal = jnp.stack([
        jnp.log(o_ce + 1e-10),
        m_cu,
        jax.nn.softplus(beta_ce),
        jax.nn.softplus(beta_cu),
    ]).astype(jnp.float32)

    g, binary = pl.pallas_call(
        functools.partial(_gate_kernel, k=k, inv_d=1.0 / dv),
        in_specs=[
            pl.BlockSpec(memory_space=pltpu.SMEM),
            pl.BlockSpec((bv, t_tc), lambda: (0, 0)),
            pl.BlockSpec((bv, t_tc), lambda: (0, 0)),
            pl.BlockSpec((bv, t_sc, _LANES), lambda: (0, 0, 0)),
            pl.BlockSpec((bv, t_sc, _LANES), lambda: (0, 0, 0)),
        ],
        out_specs=[
            pl.BlockSpec((bv, tv), lambda: (0, 0)),
            pl.BlockSpec((bv, tv), lambda: (0, 0)),
        ],
        out_shape=[
            jax.ShapeDtypeStruct((bv, tv), jnp.float32),
            jax.ShapeDtypeStruct((bv, tv), jnp.float32),
        ],
    )(scal, dst_tc, dch_tc, dst_p, dch_p)

    return (g, binary)


# speedup vs baseline: 1.1670x; 1.1670x over previous
import functools

import jax
import jax.numpy as jnp
from jax.experimental import pallas as pl
from jax.experimental.pallas import tpu as pltpu

_CAPACITY = 0.5
_BLK_T = 256


def _moments_kernel(a_ref, p_ref, dst_ref, dch_ref, *, inv_d):
    a = a_ref[...]
    p = p_ref[...]
    d = a - p
    dst_ref[...] = (jnp.sum(a * a, axis=-1) * inv_d)[:, None, :]
    dch_ref[...] = (jnp.sum(d * d, axis=-1) * inv_d)[:, None, :]


def _select_topk_mask(g, k):
    b, t = g.shape
    bits = jax.lax.bitcast_convert_type(g, jnp.int32)

    def vbody(_, carry):
        lo, hi = carry
        mid = lo + ((hi - lo) >> 1)
        cnt = jnp.sum((bits >= mid).astype(jnp.int32), axis=1, keepdims=True)
        feas = cnt >= k
        return jnp.where(feas, mid, lo), jnp.where(feas, hi, mid)

    lo0 = jnp.zeros((b, 1), jnp.int32)
    hi0 = jnp.full((b, 1), jnp.int32(0x40000001))
    tbits, _ = jax.lax.fori_loop(0, 31, vbody, (lo0, hi0))

    gt = bits > tbits
    eq = bits == tbits
    need = k - jnp.sum(gt.astype(jnp.int32), axis=1, keepdims=True)
    iota = jax.lax.broadcasted_iota(jnp.int32, (b, t), 1)
    eqi = eq.astype(jnp.int32)

    def ibody(_, carry):
        lo, hi = carry
        mid = lo + ((hi - lo) >> 1)
        cnt = jnp.sum(eqi * (iota < mid).astype(jnp.int32), axis=1,
                      keepdims=True)
        geq = cnt >= need
        return jnp.where(geq, lo, mid), jnp.where(geq, mid, hi)

    lo0 = jnp.zeros((b, 1), jnp.int32)
    hi0 = jnp.full((b, 1), jnp.int32(t))
    _, jstar = jax.lax.fori_loop(0, 14, ibody, (lo0, hi0))

    return (gt | (eq & (iota < jstar))).astype(jnp.float32)


def _gate_kernel(scal_ref, dst_ref, dch_ref, g_ref, bin_ref, *, k):
    dst = dst_ref[...]
    dch = dch_ref[...]
    log_oce = scal_ref[0]
    m_cu = scal_ref[1]
    bce_pos = scal_ref[2]
    bcu_pos = scal_ref[3]

    ce = dst - (dch - log_oce)
    ma = jnp.mean(dst)
    cu = dst - m_cu * ma
    s_ce = jax.nn.sigmoid(bce_pos * ce)
    s_cu = jax.nn.sigmoid(bcu_pos * cu)
    g = s_ce + s_cu - s_ce * s_cu
    g_ref[...] = g
    bin_ref[...] = _select_topk_mask(g, k)


def kernel(actual_residual, predicted_residual, o_ce, m_cu, beta_ce, beta_cu):
    bv, tv, dv = actual_residual.shape
    k = max(1, int(tv * _CAPACITY))

    blk = 1024
    dst, dch = pl.pallas_call(
        functools.partial(_moments_kernel, inv_d=1.0 / dv),
        grid=(bv, tv // blk),
        in_specs=[
            pl.BlockSpec((1, blk, dv), lambda b, i: (b, i, 0)),
            pl.BlockSpec((1, blk, dv), lambda b, i: (b, i, 0)),
        ],
        out_specs=[
            pl.BlockSpec((1, 1, blk), lambda b, i: (b, 0, i)),
            pl.BlockSpec((1, 1, blk), lambda b, i: (b, 0, i)),
        ],
        out_shape=[
            jax.ShapeDtypeStruct((bv, 1, tv), jnp.float32),
            jax.ShapeDtypeStruct((bv, 1, tv), jnp.float32),
        ],
        compiler_params=pltpu.CompilerParams(
            dimension_semantics=("parallel", "parallel"),
        ),
    )(actual_residual, predicted_residual)
    dst = dst.reshape(bv, tv)
    dch = dch.reshape(bv, tv)

    scal = jnp.stack([
        jnp.log(o_ce + 1e-10),
        m_cu,
        jax.nn.softplus(beta_ce),
        jax.nn.softplus(beta_cu),
    ]).astype(jnp.float32)

    g, binary = pl.pallas_call(
        functools.partial(_gate_kernel, k=k),
        in_specs=[
            pl.BlockSpec(memory_space=pltpu.SMEM),
            pl.BlockSpec((bv, tv), lambda: (0, 0)),
            pl.BlockSpec((bv, tv), lambda: (0, 0)),
        ],
        out_specs=[
            pl.BlockSpec((bv, tv), lambda: (0, 0)),
            pl.BlockSpec((bv, tv), lambda: (0, 0)),
        ],
        out_shape=[
            jax.ShapeDtypeStruct((bv, tv), jnp.float32),
            jax.ShapeDtypeStruct((bv, tv), jnp.float32),
        ],
    )(scal, dst, dch)

    return (g, binary)


# speedup vs baseline: 1.2344x; 1.0578x over previous
import functools

import jax
import jax.numpy as jnp
from jax.experimental import pallas as pl
from jax.experimental.pallas import tpu as pltpu

_CAPACITY = 0.5
_BLK_T = 128


def _moments_kernel(a_ref, p_ref, dst_ref, dch_ref, *, inv_d):
    a = a_ref[...]
    p = p_ref[...]
    d = a - p
    dst_ref[...] = jnp.sum(a * a, axis=-1) * inv_d
    dch_ref[...] = jnp.sum(d * d, axis=-1) * inv_d


def _select_topk_mask(g, k):
    b, t = g.shape
    bits = jax.lax.bitcast_convert_type(g, jnp.int32)

    def vbody(_, carry):
        lo, hi = carry
        mid = lo + ((hi - lo) >> 1)
        cnt = jnp.sum((bits >= mid).astype(jnp.int32), axis=1, keepdims=True)
        feas = cnt >= k
        return jnp.where(feas, mid, lo), jnp.where(feas, hi, mid)

    lo0 = jnp.zeros((b, 1), jnp.int32)
    hi0 = jnp.full((b, 1), jnp.int32(0x40000001))
    tbits, _ = jax.lax.fori_loop(0, 31, vbody, (lo0, hi0))

    gt = bits > tbits
    eq = bits == tbits
    need = k - jnp.sum(gt.astype(jnp.int32), axis=1, keepdims=True)
    iota = jax.lax.broadcasted_iota(jnp.int32, (b, t), 1)
    eqi = eq.astype(jnp.int32)

    def ibody(_, carry):
        lo, hi = carry
        mid = lo + ((hi - lo) >> 1)
        cnt = jnp.sum(eqi * (iota < mid).astype(jnp.int32), axis=1,
                      keepdims=True)
        geq = cnt >= need
        return jnp.where(geq, lo, mid), jnp.where(geq, mid, hi)

    lo0 = jnp.zeros((b, 1), jnp.int32)
    hi0 = jnp.full((b, 1), jnp.int32(t))
    _, jstar = jax.lax.fori_loop(0, 14, ibody, (lo0, hi0))

    return (gt | (eq & (iota < jstar))).astype(jnp.float32)


def _gate_kernel(scal_ref, dst_ref, dch_ref, g_ref, bin_ref, *, k):
    dst = dst_ref[...]
    dch = dch_ref[...]
    log_oce = scal_ref[0]
    m_cu = scal_ref[1]
    bce_pos = scal_ref[2]
    bcu_pos = scal_ref[3]

    ce = dst - (dch - log_oce)
    ma = jnp.mean(dst)
    cu = dst - m_cu * ma
    s_ce = jax.nn.sigmoid(bce_pos * ce)
    s_cu = jax.nn.sigmoid(bcu_pos * cu)
    g = s_ce + s_cu - s_ce * s_cu
    g_ref[...] = g
    bin_ref[...] = _select_topk_mask(g, k)


def kernel(actual_residual, predicted_residual, o_ce, m_cu, beta_ce, beta_cu):
    bv, tv, dv = actual_residual.shape
    k = max(1, int(tv * _CAPACITY))

    dst, dch = pl.pallas_call(
        functools.partial(_moments_kernel, inv_d=1.0 / dv),
        grid=(tv // _BLK_T,),
        in_specs=[
            pl.BlockSpec((bv, _BLK_T, dv), lambda i: (0, i, 0)),
            pl.BlockSpec((bv, _BLK_T, dv), lambda i: (0, i, 0)),
        ],
        out_specs=[
            pl.BlockSpec((bv, _BLK_T), lambda i: (0, i)),
            pl.BlockSpec((bv, _BLK_T), lambda i: (0, i)),
        ],
        out_shape=[
            jax.ShapeDtypeStruct((bv, tv), jnp.float32),
            jax.ShapeDtypeStruct((bv, tv), jnp.float32),
        ],
        compiler_params=pltpu.CompilerParams(
            dimension_semantics=("parallel",),
        ),
    )(actual_residual, predicted_residual)

    scal = jnp.stack([
        jnp.log(o_ce + 1e-10),
        m_cu,
        jax.nn.softplus(beta_ce),
        jax.nn.softplus(beta_cu),
    ]).astype(jnp.float32)

    g, binary = pl.pallas_call(
        functools.partial(_gate_kernel, k=k),
        in_specs=[
            pl.BlockSpec(memory_space=pltpu.SMEM),
            pl.BlockSpec((bv, tv), lambda: (0, 0)),
            pl.BlockSpec((bv, tv), lambda: (0, 0)),
        ],
        out_specs=[
            pl.BlockSpec((bv, tv), lambda: (0, 0)),
            pl.BlockSpec((bv, tv), lambda: (0, 0)),
        ],
        out_shape=[
            jax.ShapeDtypeStruct((bv, tv), jnp.float32),
            jax.ShapeDtypeStruct((bv, tv), jnp.float32),
        ],
    )(scal, dst, dch)

    return (g, binary)


# speedup vs baseline: 1.2372x; 1.0023x over previous
import functools

import jax
import jax.numpy as jnp
from jax.experimental import pallas as pl
from jax.experimental.pallas import tpu as pltpu

_CAPACITY = 0.5
_BLK_T = 128


def _moments_kernel(a_ref, p_ref, dst_ref, dch_ref, *, inv_d):
    a = a_ref[...]
    p = p_ref[...]
    d = a - p
    dst_ref[...] = jnp.sum(a * a, axis=-1) * inv_d
    dch_ref[...] = jnp.sum(d * d, axis=-1) * inv_d


def _select_topk_mask(g, k):
    b, t = g.shape
    bits = jax.lax.bitcast_convert_type(g, jnp.int32)

    def vbody(_, carry):
        lo, hi = carry
        mid = lo + ((hi - lo) >> 1)
        cnt = jnp.sum((bits >= mid).astype(jnp.int32), axis=1, keepdims=True)
        feas = cnt >= k
        return jnp.where(feas, mid, lo), jnp.where(feas, hi, mid)

    carry = (jnp.zeros((b, 1), jnp.int32),
             jnp.full((b, 1), jnp.int32(0x40000001)))
    for _ in range(31):
        carry = vbody(None, carry)
    tbits = carry[0]

    gt = bits > tbits
    eq = bits == tbits
    need = k - jnp.sum(gt.astype(jnp.int32), axis=1, keepdims=True)
    iota = jax.lax.broadcasted_iota(jnp.int32, (b, t), 1)
    eqi = eq.astype(jnp.int32)

    def ibody(_, carry):
        lo, hi = carry
        mid = lo + ((hi - lo) >> 1)
        cnt = jnp.sum(eqi * (iota < mid).astype(jnp.int32), axis=1,
                      keepdims=True)
        geq = cnt >= need
        return jnp.where(geq, lo, mid), jnp.where(geq, mid, hi)

    carry = (jnp.zeros((b, 1), jnp.int32),
             jnp.full((b, 1), jnp.int32(t)))
    for _ in range(14):
        carry = ibody(None, carry)
    jstar = carry[1]

    return (gt | (eq & (iota < jstar))).astype(jnp.float32)


def _gate_kernel(scal_ref, dst_ref, dch_ref, g_ref, bin_ref, *, k):
    dst = dst_ref[...]
    dch = dch_ref[...]
    log_oce = scal_ref[0]
    m_cu = scal_ref[1]
    bce_pos = scal_ref[2]
    bcu_pos = scal_ref[3]

    ce = dst - (dch - log_oce)
    ma = jnp.mean(dst)
    cu = dst - m_cu * ma
    s_ce = jax.nn.sigmoid(bce_pos * ce)
    s_cu = jax.nn.sigmoid(bcu_pos * cu)
    g = s_ce + s_cu - s_ce * s_cu
    g_ref[...] = g
    bin_ref[...] = _select_topk_mask(g, k)


def kernel(actual_residual, predicted_residual, o_ce, m_cu, beta_ce, beta_cu):
    bv, tv, dv = actual_residual.shape
    k = max(1, int(tv * _CAPACITY))

    dst, dch = pl.pallas_call(
        functools.partial(_moments_kernel, inv_d=1.0 / dv),
        grid=(tv // _BLK_T,),
        in_specs=[
            pl.BlockSpec((bv, _BLK_T, dv), lambda i: (0, i, 0)),
            pl.BlockSpec((bv, _BLK_T, dv), lambda i: (0, i, 0)),
        ],
        out_specs=[
            pl.BlockSpec((bv, _BLK_T), lambda i: (0, i)),
            pl.BlockSpec((bv, _BLK_T), lambda i: (0, i)),
        ],
        out_shape=[
            jax.ShapeDtypeStruct((bv, tv), jnp.float32),
            jax.ShapeDtypeStruct((bv, tv), jnp.float32),
        ],
        compiler_params=pltpu.CompilerParams(
            dimension_semantics=("parallel",),
        ),
    )(actual_residual, predicted_residual)

    scal = jnp.stack([
        jnp.log(o_ce + 1e-10),
        m_cu,
        jax.nn.softplus(beta_ce),
        jax.nn.softplus(beta_cu),
    ]).astype(jnp.float32)

    g, binary = pl.pallas_call(
        functools.partial(_gate_kernel, k=k),
        in_specs=[
            pl.BlockSpec(memory_space=pltpu.SMEM),
            pl.BlockSpec((bv, tv), lambda: (0, 0)),
            pl.BlockSpec((bv, tv), lambda: (0, 0)),
        ],
        out_specs=[
            pl.BlockSpec((bv, tv), lambda: (0, 0)),
            pl.BlockSpec((bv, tv), lambda: (0, 0)),
        ],
        out_shape=[
            jax.ShapeDtypeStruct((bv, tv), jnp.float32),
            jax.ShapeDtypeStruct((bv, tv), jnp.float32),
        ],
    )(scal, dst, dch)

    return (g, binary)


# speedup vs baseline: 1.2493x; 1.0098x over previous
import functools

import jax
import jax.numpy as jnp
from jax.experimental import pallas as pl
from jax.experimental.pallas import tpu as pltpu

_CAPACITY = 0.5
_BLK_T = 128


def _moments_kernel(a_ref, p_ref, dst_ref, dch_ref, *, inv_d):
    a = a_ref[...]
    p = p_ref[...]
    d = a - p
    dst_ref[...] = jnp.sum(a * a, axis=-1) * inv_d
    dch_ref[...] = jnp.sum(d * d, axis=-1) * inv_d


def _select_topk_mask(g, k):
    b, t = g.shape
    bits = jax.lax.bitcast_convert_type(g, jnp.int32)

    def vbody(_, carry):
        lo, hi = carry
        mid = lo + ((hi - lo) >> 1)
        cnt = jnp.sum((bits >= mid).astype(jnp.int32), axis=1, keepdims=True)
        feas = cnt >= k
        return jnp.where(feas, mid, lo), jnp.where(feas, hi, mid)

    carry = (jnp.zeros((b, 1), jnp.int32),
             jnp.full((b, 1), jnp.int32(0x40000001)))
    for _ in range(31):
        carry = vbody(None, carry)
    tbits = carry[0]

    gt = bits > tbits
    eq = bits == tbits
    need = k - jnp.sum(gt.astype(jnp.int32), axis=1, keepdims=True)
    iota = jax.lax.broadcasted_iota(jnp.int32, (b, t), 1)
    eqi = eq.astype(jnp.int32)

    def ibody(_, carry):
        lo, hi = carry
        mid = lo + ((hi - lo) >> 1)
        cnt = jnp.sum(eqi * (iota < mid).astype(jnp.int32), axis=1,
                      keepdims=True)
        geq = cnt >= need
        return jnp.where(geq, lo, mid), jnp.where(geq, mid, hi)

    carry = (jnp.zeros((b, 1), jnp.int32),
             jnp.full((b, 1), jnp.int32(t)))
    for _ in range(14):
        carry = ibody(None, carry)
    jstar = carry[1]

    return (gt | (eq & (iota < jstar))).astype(jnp.float32)


def _gate_kernel(scal_ref, dst_ref, dch_ref, g_ref, bin_ref, *, k):
    dst = dst_ref[...]
    dch = dch_ref[...]
    log_oce = jnp.log(scal_ref[0] + 1e-10)
    m_cu = scal_ref[1]
    bce_pos = jax.nn.softplus(scal_ref[2])
    bcu_pos = jax.nn.softplus(scal_ref[3])

    ce = dst - (dch - log_oce)
    ma = jnp.mean(dst)
    cu = dst - m_cu * ma
    s_ce = jax.nn.sigmoid(bce_pos * ce)
    s_cu = jax.nn.sigmoid(bcu_pos * cu)
    g = s_ce + s_cu - s_ce * s_cu
    g_ref[...] = g
    bin_ref[...] = _select_topk_mask(g, k)


def kernel(actual_residual, predicted_residual, o_ce, m_cu, beta_ce, beta_cu):
    bv, tv, dv = actual_residual.shape
    k = max(1, int(tv * _CAPACITY))

    dst, dch = pl.pallas_call(
        functools.partial(_moments_kernel, inv_d=1.0 / dv),
        grid=(tv // _BLK_T,),
        in_specs=[
            pl.BlockSpec((bv, _BLK_T, dv), lambda i: (0, i, 0)),
            pl.BlockSpec((bv, _BLK_T, dv), lambda i: (0, i, 0)),
        ],
        out_specs=[
            pl.BlockSpec((bv, _BLK_T), lambda i: (0, i)),
            pl.BlockSpec((bv, _BLK_T), lambda i: (0, i)),
        ],
        out_shape=[
            jax.ShapeDtypeStruct((bv, tv), jnp.float32),
            jax.ShapeDtypeStruct((bv, tv), jnp.float32),
        ],
        compiler_params=pltpu.CompilerParams(
            dimension_semantics=("parallel",),
        ),
    )(actual_residual, predicted_residual)

    scal = jnp.stack([o_ce, m_cu, beta_ce, beta_cu]).astype(jnp.float32)

    g, binary = pl.pallas_call(
        functools.partial(_gate_kernel, k=k),
        in_specs=[
            pl.BlockSpec(memory_space=pltpu.SMEM),
            pl.BlockSpec((bv, tv), lambda: (0, 0)),
            pl.BlockSpec((bv, tv), lambda: (0, 0)),
        ],
        out_specs=[
            pl.BlockSpec((bv, tv), lambda: (0, 0)),
            pl.BlockSpec((bv, tv), lambda: (0, 0)),
        ],
        out_shape=[
            jax.ShapeDtypeStruct((bv, tv), jnp.float32),
            jax.ShapeDtypeStruct((bv, tv), jnp.float32),
        ],
    )(scal, dst, dch)

    return (g, binary)
